# baseline (speedup 1.0000x reference)
import functools

import jax
import jax.numpy as jnp
from jax.experimental import pallas as pl

_NT = (((1,), (1,)), ((), ()))


def _pre_kernel(ctx_ref, cm_ref, w_ref, cn_ref, m_ref, w16_ref):
    c = ctx_ref[...]
    norms = jnp.sqrt(jnp.sum(c * c, axis=1, keepdims=True))
    cn_ref[...] = (c / jnp.clip(norms, 1e-8, None)).astype(jnp.bfloat16)
    seg = jax.lax.dot_general(cm_ref[...], c, _NT,
                              preferred_element_type=jnp.float32)
    m_ref[...] = jnp.max(seg, axis=0, keepdims=True)
    w16_ref[...] = w_ref[...].astype(jnp.bfloat16)


def _main_kernel(x_ref, cn_ref, m_ref, w16_ref, b_ref, out_ref, *,
                 n_total, n_chunks):
    n = pl.program_id(1)
    xb16 = x_ref[0].astype(jnp.bfloat16)
    n_ctx = cn_ref.shape[0]
    l_out = w16_ref.shape[0]
    cc = n_ctx // n_chunks
    oc = l_out // n_chunks

    cmaxs, cmvals = [], []
    for j in range(n_chunks):
        sc = jax.lax.dot_general(xb16, cn_ref[j * cc:(j + 1) * cc, :], _NT,
                                 preferred_element_type=jnp.float32)
        cmax = jnp.max(sc, axis=1, keepdims=True)
        cmval = jnp.max(
            jnp.where(sc == cmax, m_ref[:, j * cc:(j + 1) * cc], -jnp.inf),
            axis=1, keepdims=True)
        cmaxs.append(cmax)
        cmvals.append(cmval)
    gmax = functools.reduce(jnp.maximum, cmaxs)
    mval = functools.reduce(
        jnp.maximum,
        [jnp.where(cm == gmax, mv, -jnp.inf) for cm, mv in zip(cmaxs, cmvals)])
    act = jax.nn.sigmoid(mval) * (0.5 / n_total)

    first = n == 0
    for j in range(n_chunks):
        sl = slice(j * oc, (j + 1) * oc)
        h = jax.lax.dot_general(xb16, w16_ref[sl, :], _NT,
                                preferred_element_type=jnp.float32)
        h = h + b_ref[:, sl]
        g = h * (1.0 + jax.lax.erf(h * 0.7071067811865476))
        contrib = g * act

        @pl.when(first)
        def _(contrib=contrib, sl=sl):
            out_ref[:, sl] = contrib

        @pl.when(jnp.logical_not(first))
        def _(contrib=contrib, sl=sl):
            out_ref[:, sl] += contrib


def kernel(x, ctx_mod, context, W, b):
    N, B, L = x.shape
    n_ctx = context.shape[0]

    cn16, m, w16 = pl.pallas_call(
        _pre_kernel,
        out_shape=[
            jax.ShapeDtypeStruct((n_ctx, L), jnp.bfloat16),
            jax.ShapeDtypeStruct((1, n_ctx), jnp.float32),
            jax.ShapeDtypeStruct((L, L), jnp.bfloat16),
        ],
    )(context, ctx_mod, W)

    b2 = b.reshape(1, L)

    tb = 1024 if B % 1024 == 0 else B
    nb = B // tb
    out = pl.pallas_call(
        functools.partial(_main_kernel, n_total=N,
                          n_chunks=4 if (n_ctx % 4 == 0 and L % 4 == 0) else 1),
        grid=(nb, N),
        in_specs=[
            pl.BlockSpec((1, tb, L), lambda bi, n: (n, bi, 0)),
            pl.BlockSpec((n_ctx, L), lambda bi, n: (0, 0)),
            pl.BlockSpec((1, n_ctx), lambda bi, n: (0, 0)),
            pl.BlockSpec((L, L), lambda bi, n: (0, 0)),
            pl.BlockSpec((1, L), lambda bi, n: (0, 0)),
        ],
        out_specs=pl.BlockSpec((tb, L), lambda bi, n: (bi, 0)),
        out_shape=jax.ShapeDtypeStruct((B, L), jnp.float32),
    )(x, cn16, m, w16, b2)
    return out


# speedup vs baseline: 1.2924x; 1.2924x over previous
import functools

import jax
import jax.numpy as jnp
from jax.experimental import pallas as pl
from jax.experimental.pallas import tpu as pltpu

_NT = (((1,), (1,)), ((), ()))


def _pre_kernel(ctx_ref, cm_ref, w_ref, cn_ref, m_ref, w16_ref):
    c = ctx_ref[...]
    norms = jnp.sqrt(jnp.sum(c * c, axis=1, keepdims=True))
    cn_ref[...] = (c / jnp.clip(norms, 1e-8, None)).astype(jnp.bfloat16)
    seg = jax.lax.dot_general(cm_ref[...], c, _NT,
                              preferred_element_type=jnp.float32)
    m_ref[...] = jnp.max(seg, axis=0, keepdims=True)
    w16_ref[...] = w_ref[...].astype(jnp.bfloat16)


def _main_kernel(x_ref, cn_ref, m_ref, w16_ref, b_ref, out_ref,
                 s0, h0, s1, h1, *, n_total):
    k = pl.program_id(0)

    def compute(s_buf, h_buf):
        xb16 = x_ref[0].astype(jnp.bfloat16)
        s_buf[...] = jax.lax.dot_general(xb16, cn_ref[...], _NT,
                                         preferred_element_type=jnp.float32)
        h_buf[...] = jax.lax.dot_general(xb16, w16_ref[...], _NT,
                                         preferred_element_type=jnp.float32)

    def epilogue(s_buf, h_buf):
        s = s_buf[...]
        rowmax = jnp.max(s, axis=1, keepdims=True)
        mval = jnp.max(jnp.where(s == rowmax, m_ref[...], -jnp.inf),
                       axis=1, keepdims=True)
        act = jax.nn.sigmoid(mval) * (0.5 / n_total)
        h = h_buf[...] + b_ref[...]
        g = h * (1.0 + jax.lax.erf(h * 0.7071067811865476))
        contrib = g * act
        np_ = jax.lax.rem(k - 1, n_total)
        prev = out_ref[...]
        out_ref[...] = contrib + jnp.where(np_ > 0, prev, 0.0)

    @pl.when(jax.lax.rem(k, 2) == 0)
    def _():
        compute(s0, h0)
        epilogue(s1, h1)

    @pl.when(jax.lax.rem(k, 2) == 1)
    def _():
        compute(s1, h1)
        epilogue(s0, h0)


def kernel(x, ctx_mod, context, W, b):
    N, B, L = x.shape
    n_ctx = context.shape[0]

    cn16, m, w16 = pl.pallas_call(
        _pre_kernel,
        out_shape=[
            jax.ShapeDtypeStruct((n_ctx, L), jnp.bfloat16),
            jax.ShapeDtypeStruct((1, n_ctx), jnp.float32),
            jax.ShapeDtypeStruct((L, L), jnp.bfloat16),
        ],
    )(context, ctx_mod, W)

    b2 = b.reshape(1, L)

    tb = 1024 if B % 1024 == 0 else B
    nb = B // tb
    n_steps = nb * N + 1

    def x_index(k):
        km = jnp.minimum(k, n_steps - 2)
        return (jax.lax.rem(km, N), km // N, 0)

    out = pl.pallas_call(
        functools.partial(_main_kernel, n_total=N),
        grid=(n_steps,),
        in_specs=[
            pl.BlockSpec((1, tb, L), x_index),
            pl.BlockSpec((n_ctx, L), lambda k: (0, 0)),
            pl.BlockSpec((1, n_ctx), lambda k: (0, 0)),
            pl.BlockSpec((L, L), lambda k: (0, 0)),
            pl.BlockSpec((1, L), lambda k: (0, 0)),
        ],
        out_specs=pl.BlockSpec(
            (tb, L), lambda k: (jnp.maximum(k - 1, 0) // N, 0)),
        out_shape=jax.ShapeDtypeStruct((B, L), jnp.float32),
        scratch_shapes=[
            pltpu.VMEM((tb, n_ctx), jnp.float32),
            pltpu.VMEM((tb, L), jnp.float32),
            pltpu.VMEM((tb, n_ctx), jnp.float32),
            pltpu.VMEM((tb, L), jnp.float32),
        ],
    )(x, cn16, m, w16, b2)
    return out


# speedup vs baseline: 1.4877x; 1.1511x over previous
import functools

import jax
import jax.numpy as jnp
from jax.experimental import pallas as pl

_NT = (((1,), (1,)), ((), ()))


def _pre_kernel(ctx_ref, cm_ref, w_ref, cn_ref, m_ref, w16_ref):
    c = ctx_ref[...]
    norms = jnp.sqrt(jnp.sum(c * c, axis=1, keepdims=True))
    cn_ref[...] = (c / jnp.clip(norms, 1e-8, None)).astype(jnp.bfloat16)
    seg = jax.lax.dot_general(cm_ref[...], c, _NT,
                              preferred_element_type=jnp.float32)
    m_ref[...] = jnp.max(seg, axis=0, keepdims=True)
    w16_ref[...] = w_ref[...].astype(jnp.bfloat16)


def _main_kernel(x_ref, cn_ref, m_ref, w16_ref, b_ref, out_ref, *, n_total):
    tb = out_ref.shape[0]
    l_dim = x_ref.shape[2]
    xb16 = x_ref[...].reshape(n_total * tb, l_dim).astype(jnp.bfloat16)
    s = jax.lax.dot_general(xb16, cn_ref[...], _NT,
                            preferred_element_type=jnp.float32)
    rowmax = jnp.max(s, axis=1, keepdims=True)
    mval = jnp.max(jnp.where(s == rowmax, m_ref[...], -jnp.inf),
                   axis=1, keepdims=True)
    act = jax.nn.sigmoid(mval) * (0.5 / n_total)
    h = jax.lax.dot_general(xb16, w16_ref[...], _NT,
                            preferred_element_type=jnp.float32) + b_ref[...]
    g = h * (1.0 + jax.lax.erf(h * 0.7071067811865476))
    contrib = g * act
    parts = [contrib[i * tb:(i + 1) * tb, :] for i in range(n_total)]
    while len(parts) > 1:
        parts = [parts[i] + parts[i + 1] for i in range(0, len(parts), 2)] + \
            (parts[-1:] if len(parts) % 2 else [])
    out_ref[...] = parts[0]


def kernel(x, ctx_mod, context, W, b):
    N, B, L = x.shape
    n_ctx = context.shape[0]

    cn16, m, w16 = pl.pallas_call(
        _pre_kernel,
        out_shape=[
            jax.ShapeDtypeStruct((n_ctx, L), jnp.bfloat16),
            jax.ShapeDtypeStruct((1, n_ctx), jnp.float32),
            jax.ShapeDtypeStruct((L, L), jnp.bfloat16),
        ],
    )(context, ctx_mod, W)

    b2 = b.reshape(1, L)

    tb = 128 if B % 128 == 0 else B
    out = pl.pallas_call(
        functools.partial(_main_kernel, n_total=N),
        grid=(B // tb,),
        in_specs=[
            pl.BlockSpec((N, tb, L), lambda bi: (0, bi, 0)),
            pl.BlockSpec((n_ctx, L), lambda bi: (0, 0)),
            pl.BlockSpec((1, n_ctx), lambda bi: (0, 0)),
            pl.BlockSpec((L, L), lambda bi: (0, 0)),
            pl.BlockSpec((1, L), lambda bi: (0, 0)),
        ],
        out_specs=pl.BlockSpec((tb, L), lambda bi: (bi, 0)),
        out_shape=jax.ShapeDtypeStruct((B, L), jnp.float32),
    )(x, cn16, m, w16, b2)
    return out
